# baseline (speedup 1.0000x reference)
import functools

import jax
import jax.numpy as jnp
from jax import lax
from jax.experimental import pallas as pl
from jax.experimental.pallas import tpu as pltpu
from jax.experimental.pallas import tpu_sc as plsc

D = 1024
DFF = 2816
E = 8
T = 32
BF = 1408
NBF = DFF // BF
L = 16
NCHUNK = D // L


def _router_body(x_hbm, gw_hbm, comb_hbm, xv, gwv, combv):
    t = lax.axis_index("s") * 2 + lax.axis_index("c")
    pltpu.sync_copy(x_hbm.at[t], xv)
    pltpu.sync_copy(gw_hbm, gwv)
    lanes = lax.iota(jnp.int32, L)
    lv = jnp.full((L,), -1e30, dtype=jnp.float32)
    for e in range(E):
        acc = jnp.zeros((L,), jnp.float32)
        for k in range(NCHUNK):
            acc = acc + xv[pl.ds(k * L, L)] * gwv[e, pl.ds(k * L, L)]
        lv = jnp.where(lanes == e, jnp.sum(acc), lv)
    p = jnp.exp(lv - jnp.max(lv))
    m1 = jnp.max(p)
    i1 = jnp.min(jnp.where(p == m1, lanes, L))
    mask1 = lanes == i1
    p2 = jnp.where(mask1, -1.0, p)
    m2 = jnp.max(p2)
    i2 = jnp.min(jnp.where(p2 == m2, lanes, L))
    mask = mask1 | (lanes == i2)
    pm = jnp.where(mask, p, 0.0)
    combv[...] = pm / jnp.sum(pm)
    pltpu.sync_copy(combv, comb_hbm.at[t])


_sc_router = functools.partial(
    pl.kernel,
    out_type=jax.ShapeDtypeStruct((T, L), jnp.float32),
    compiler_params=pltpu.CompilerParams(needs_layout_passes=False),
    mesh=plsc.VectorSubcoreMesh(core_axis_name="c", subcore_axis_name="s"),
    scratch_types=[
        pltpu.VMEM((D,), jnp.float32),
        pltpu.VMEM((E, D), jnp.float32),
        pltpu.VMEM((L,), jnp.float32),
    ],
)(_router_body)


def _moe_body(x_ref, comb_ref, wg_ref, wu_ref, wd_ref, out_ref):
    e = pl.program_id(0)
    j = pl.program_id(1)

    @pl.when((e == 0) & (j == 0))
    def _init():
        out_ref[...] = jnp.zeros_like(out_ref)

    xv = x_ref[...].astype(jnp.bfloat16)
    g = jax.lax.dot_general(
        xv, wg_ref[0].astype(jnp.bfloat16), (((1,), (1,)), ((), ())),
        preferred_element_type=jnp.float32)
    u = jax.lax.dot_general(
        xv, wu_ref[0].astype(jnp.bfloat16), (((1,), (1,)), ((), ())),
        preferred_element_type=jnp.float32)
    act = (g * jax.nn.sigmoid(g) * u).astype(jnp.bfloat16)
    part = jax.lax.dot_general(
        act, wd_ref[0].astype(jnp.bfloat16), (((1,), (1,)), ((), ())),
        preferred_element_type=jnp.float32)
    sel = (jax.lax.broadcasted_iota(jnp.int32, (L, 1), 0) == e).astype(
        jnp.float32)
    scale = jax.lax.dot_general(
        comb_ref[...], sel, (((1,), (0,)), ((), ())),
        preferred_element_type=jnp.float32)
    out_ref[...] += part * scale


def kernel(x, gate_w, Wg, Wu, Wd):
    x2d = x.reshape(T, D)
    comb = _sc_router(x2d, gate_w)
    out = pl.pallas_call(
        _moe_body,
        grid=(E, NBF),
        in_specs=[
            pl.BlockSpec((T, D), lambda e, j: (0, 0)),
            pl.BlockSpec((T, L), lambda e, j: (0, 0)),
            pl.BlockSpec((1, BF, D), lambda e, j: (e, j, 0)),
            pl.BlockSpec((1, BF, D), lambda e, j: (e, j, 0)),
            pl.BlockSpec((1, D, BF), lambda e, j: (e, 0, j)),
        ],
        out_specs=pl.BlockSpec((T, D), lambda e, j: (0, 0)),
        out_shape=jax.ShapeDtypeStruct((T, D), jnp.float32),
    )(x2d, comb, Wg, Wu, Wd)
    return out.reshape(x.shape)


# speedup vs baseline: 1.2543x; 1.2543x over previous
import jax
import jax.numpy as jnp
from jax.experimental import pallas as pl
from jax.experimental.pallas import tpu as pltpu

D = 1024
DFF = 2816
E = 8
T = 32
BF = 1408
NBF = DFF // BF


def _moe_body(x_ref, gw_ref, wg_ref, wu_ref, wd_ref, out_ref, comb_ref):
    e = pl.program_id(0)
    j = pl.program_id(1)

    @pl.when((e == 0) & (j == 0))
    def _router():
        xv = x_ref[...]
        logits = jax.lax.dot_general(
            xv, gw_ref[...], (((1,), (1,)), ((), ())),
            preferred_element_type=jnp.float32)
        p = jnp.exp(logits - jnp.max(logits, axis=1, keepdims=True))
        idx = jax.lax.broadcasted_iota(jnp.int32, (T, E), 1)
        m1 = jnp.max(p, axis=1, keepdims=True)
        i1 = jnp.min(jnp.where(p == m1, idx, E), axis=1, keepdims=True)
        mask1 = idx == i1
        p_wo = jnp.where(mask1, -jnp.inf, p)
        m2 = jnp.max(p_wo, axis=1, keepdims=True)
        i2 = jnp.min(jnp.where(p_wo == m2, idx, E), axis=1, keepdims=True)
        mask = mask1 | (idx == i2)
        pm = jnp.where(mask, p, 0.0)
        comb_ref[...] = pm / jnp.sum(pm, axis=1, keepdims=True)
        out_ref[...] = jnp.zeros_like(out_ref)

    xv = x_ref[...].astype(jnp.bfloat16)
    g = jax.lax.dot_general(
        xv, wg_ref[0].astype(jnp.bfloat16), (((1,), (1,)), ((), ())),
        preferred_element_type=jnp.float32)
    u = jax.lax.dot_general(
        xv, wu_ref[0].astype(jnp.bfloat16), (((1,), (1,)), ((), ())),
        preferred_element_type=jnp.float32)
    act = (g * jax.nn.sigmoid(g) * u).astype(jnp.bfloat16)
    part = jax.lax.dot_general(
        act, wd_ref[0].astype(jnp.bfloat16), (((1,), (1,)), ((), ())),
        preferred_element_type=jnp.float32)
    sel = (jax.lax.broadcasted_iota(jnp.int32, (E, 1), 0) == e).astype(
        jnp.float32)
    scale = jax.lax.dot_general(
        comb_ref[...], sel, (((1,), (0,)), ((), ())),
        preferred_element_type=jnp.float32)
    out_ref[...] += part * scale


def kernel(x, gate_w, Wg, Wu, Wd):
    x2d = x.reshape(T, D)
    out = pl.pallas_call(
        _moe_body,
        grid=(E, NBF),
        in_specs=[
            pl.BlockSpec((T, D), lambda e, j: (0, 0)),
            pl.BlockSpec((E, D), lambda e, j: (0, 0)),
            pl.BlockSpec((1, BF, D), lambda e, j: (e, j, 0)),
            pl.BlockSpec((1, BF, D), lambda e, j: (e, j, 0)),
            pl.BlockSpec((1, D, BF), lambda e, j: (e, 0, j)),
        ],
        out_specs=pl.BlockSpec((T, D), lambda e, j: (0, 0)),
        out_shape=jax.ShapeDtypeStruct((T, D), jnp.float32),
        scratch_shapes=[pltpu.VMEM((T, E), jnp.float32)],
    )(x2d, gate_w, Wg, Wu, Wd)
    return out.reshape(x.shape)
